# baseline (speedup 1.0000x reference)
import functools

import jax
import jax.numpy as jnp
from jax import lax
from jax.experimental import pallas as pl
from jax.experimental.pallas import tpu as pltpu

B, N, T = 16, 20000, 30
EVAL_NUM = 6
POS_NUM = 10
D = T * 2
BIG = 2**30
NEG = -jnp.inf


def _argmax_rows(work, lin):
    m = jnp.max(work, axis=1)
    cand = jnp.where(work == m[:, None], lin, BIG)
    idx = jnp.min(cand, axis=1)
    oh = lin == idx[:, None]
    return m, idx, oh


def _ext(arr, oh):
    return jnp.sum(jnp.where(oh, arr, 0.0), axis=1)


def _smooth_l1_sum(diff):
    d = jnp.abs(diff)
    return jnp.where(d < 1.0, 0.5 * d * d, d - 0.5)


def _compute_body(x_ref, y_ref, cls_ref, cent_ref, tp_ref, scal_ref, idx_ref):
    x = x_ref[:, :]
    y = y_ref[:, :]
    cls = cls_ref[:, :]
    cent = cent_ref[:, :]
    tx = tp_ref[:, 0]
    ty = tp_ref[:, 1]

    lin = lax.broadcasted_iota(jnp.int32, (B, N), 1)
    lane16 = lax.broadcasted_iota(jnp.int32, (B, 16), 1)
    lane8 = lax.broadcasted_iota(jnp.int32, (B, 8), 1)

    work = cls
    idxs_acc = jnp.zeros((B, 16), jnp.int32)
    point_sum = jnp.zeros((B,), jnp.float32)
    class_sum = jnp.zeros((B,), jnp.float32)
    cent_sum = jnp.zeros((B,), jnp.float32)
    neg_top_sum = jnp.zeros((B,), jnp.float32)
    for k in range(POS_NUM):
        v, idx, oh = _argmax_rows(work, lin)
        xk = _ext(x, oh)
        yk = _ext(y, oh)
        ck = _ext(cent, oh)
        idxs_acc = idxs_acc + jnp.where(lane16 == k, idx[:, None], 0)
        point_sum = point_sum + _smooth_l1_sum(xk - tx) + _smooth_l1_sum(yk - ty)
        class_sum = class_sum - jnp.log(jnp.clip(v, 1e-7, 1.0 - 1e-7))
        dk = jnp.sqrt((xk - tx) ** 2 + (yk - ty) ** 2 + 1e-12)
        tgt = jnp.where(dk >= 2.0, 0.0, 1.0 - jnp.sqrt(dk / 2.0))
        p = jnp.clip(ck, 1e-7, 1.0 - 1e-7)
        cent_sum = cent_sum - (tgt * jnp.log(p) + (1.0 - tgt) * jnp.log(1.0 - p))
        neg_top_sum = neg_top_sum - jnp.log(jnp.clip(1.0 - v, 1e-7, 1.0))
        work = jnp.where(oh, NEG, work)

    point_loss = point_sum / (POS_NUM * 2)
    class_loss = class_sum / POS_NUM
    centerness_loss = cent_sum / POS_NUM
    neg_all = jnp.sum(-jnp.log(jnp.clip(1.0 - cls, 1e-7, 1.0)), axis=1)
    neg_class_loss = (neg_all - neg_top_sum) / (N - POS_NUM)
    partial_loss = point_loss + class_loss + centerness_loss + neg_class_loss

    comb = cls * cent
    _, _, ohb = _argmax_rows(comb, lin)
    xb = _ext(x, ohb)
    yb = _ext(y, ohb)
    de = jnp.sqrt((xb - tx) ** 2 + (yb - ty) ** 2 + 1e-12)

    scores = comb
    scal = jnp.where(lane8 == 0, partial_loss[:, None], 0.0)
    scal = scal + jnp.where(lane8 == 1, de[:, None], 0.0)
    for k in range(EVAL_NUM):
        _, idx, oh = _argmax_rows(scores, lin)
        pk = _ext(comb, oh)
        xi = _ext(x, oh)
        yi = _ext(y, oh)
        idxs_acc = idxs_acc + jnp.where(
            lane16 == (POS_NUM + k), idx[:, None], 0)
        scal = scal + jnp.where(lane8 == (2 + k), pk[:, None], 0.0)
        d2 = (x - xi[:, None]) ** 2 + (y - yi[:, None]) ** 2
        scores = jnp.where(d2 + 1e-12 < 4.0, NEG, scores)

    scal_ref[:, :] = scal
    idx_ref[:, :] = idxs_acc


def _gather_body(idx_ref, traj_ref, rows_ref, sem):
    b = pl.program_id(0)
    copies = []
    for j in range(16):
        c = pltpu.make_async_copy(
            traj_ref.at[b, idx_ref[0, 0, j]], rows_ref.at[0, j], sem)
        c.start()
        copies.append(c)
    for c in copies:
        c.wait()


def _finish_body(rows_ref, gt_ref, scal_ref, out_ref):
    rows = rows_ref[:, :, :, :]
    gt = gt_ref[:, :, :]
    sl1 = _smooth_l1_sum(rows - gt[:, None, :, :])
    rmask = lax.broadcasted_iota(jnp.int32, (B, 16, 1, 1), 1) < POS_NUM
    traj_loss = jnp.sum(jnp.where(rmask, sl1, 0.0), axis=(1, 2, 3)) / (POS_NUM * D)
    total = scal_ref[:, 0] + traj_loss
    out_ref[:, :] = jnp.reshape(jnp.sum(total) / B, (1, 1))


@jax.jit
def kernel(outputs_coord, outputs_class, outputs_traj, outputs_centerness,
           gt_points):
    interpret = False
    sel = jnp.array([[1.0, 0.0], [0.0, 1.0]], jnp.float32)
    x = jnp.sum(outputs_coord * sel[0], axis=-1)
    y = jnp.sum(outputs_coord * sel[1], axis=-1)
    tp = gt_points[:, -1, :]

    scal, gidx = pl.pallas_call(
        _compute_body,
        out_shape=(
            jax.ShapeDtypeStruct((B, 8), jnp.float32),
            jax.ShapeDtypeStruct((B, 16), jnp.int32),
        ),
        interpret=interpret,
    )(x, y, outputs_class, outputs_centerness, tp)

    gidx3 = gidx.reshape(B, 1, 16)
    rows = pl.pallas_call(
        _gather_body,
        grid=(B,),
        in_specs=[
            pl.BlockSpec((1, 1, 16), lambda b: (b, 0, 0),
                         memory_space=pltpu.SMEM),
            pl.BlockSpec(memory_space=pl.MemorySpace.ANY),
        ],
        out_specs=pl.BlockSpec((1, 16, T, 2), lambda b: (b, 0, 0, 0)),
        out_shape=jax.ShapeDtypeStruct((B, 16, T, 2), jnp.float32),
        scratch_shapes=[pltpu.SemaphoreType.DMA],
        interpret=interpret,
    )(gidx3, outputs_traj)

    loss = pl.pallas_call(
        _finish_body,
        out_shape=jax.ShapeDtypeStruct((1, 1), jnp.float32),
        interpret=interpret,
    )(rows, gt_points, scal)

    return (loss[0, 0], scal[:, 1], rows[:, POS_NUM:], scal[:, 2:8])


# speedup vs baseline: 13.7819x; 13.7819x over previous
import functools

import jax
import jax.numpy as jnp
from jax import lax
from jax.experimental import pallas as pl
from jax.experimental.pallas import tpu as pltpu

B, N, T = 16, 20000, 30
EVAL_NUM = 6
POS_NUM = 10
D = T * 2
BIG = 2**30
NEG = -jnp.inf


def _argmax_rows(work, lin):
    m = jnp.max(work, axis=1)
    cand = jnp.where(work == m[:, None], lin, BIG)
    idx = jnp.min(cand, axis=1)
    oh = lin == idx[:, None]
    return m, idx, oh


def _ext(arr, oh):
    return jnp.sum(jnp.where(oh, arr, 0.0), axis=1)


def _smooth_l1_sum(diff):
    d = jnp.abs(diff)
    return jnp.where(d < 1.0, 0.5 * d * d, d - 0.5)


def _compute_body(x_ref, y_ref, cls_ref, cent_ref, tp_ref, scal_ref, idx_ref):
    x = x_ref[:, :]
    y = y_ref[:, :]
    cls = cls_ref[:, :]
    cent = cent_ref[:, :]
    tx = tp_ref[:, 0]
    ty = tp_ref[:, 1]

    lin = lax.broadcasted_iota(jnp.int32, (B, N), 1)
    sub16 = lax.broadcasted_iota(jnp.int32, (16, B), 0)
    lane8 = lax.broadcasted_iota(jnp.int32, (B, 8), 1)

    work = cls
    idxs_acc = jnp.zeros((16, B), jnp.int32)
    point_sum = jnp.zeros((B,), jnp.float32)
    class_sum = jnp.zeros((B,), jnp.float32)
    cent_sum = jnp.zeros((B,), jnp.float32)
    neg_top_sum = jnp.zeros((B,), jnp.float32)
    for k in range(POS_NUM):
        v, idx, oh = _argmax_rows(work, lin)
        xk = _ext(x, oh)
        yk = _ext(y, oh)
        ck = _ext(cent, oh)
        idxs_acc = idxs_acc + jnp.where(sub16 == k, idx[None, :], 0)
        point_sum = point_sum + _smooth_l1_sum(xk - tx) + _smooth_l1_sum(yk - ty)
        class_sum = class_sum - jnp.log(jnp.clip(v, 1e-7, 1.0 - 1e-7))
        dk = jnp.sqrt((xk - tx) ** 2 + (yk - ty) ** 2 + 1e-12)
        tgt = jnp.where(dk >= 2.0, 0.0, 1.0 - jnp.sqrt(dk / 2.0))
        p = jnp.clip(ck, 1e-7, 1.0 - 1e-7)
        cent_sum = cent_sum - (tgt * jnp.log(p) + (1.0 - tgt) * jnp.log(1.0 - p))
        neg_top_sum = neg_top_sum - jnp.log(jnp.clip(1.0 - v, 1e-7, 1.0))
        work = jnp.where(oh, NEG, work)

    point_loss = point_sum / (POS_NUM * 2)
    class_loss = class_sum / POS_NUM
    centerness_loss = cent_sum / POS_NUM
    neg_all = jnp.sum(-jnp.log(jnp.clip(1.0 - cls, 1e-7, 1.0)), axis=1)
    neg_class_loss = (neg_all - neg_top_sum) / (N - POS_NUM)
    partial_loss = point_loss + class_loss + centerness_loss + neg_class_loss

    comb = cls * cent
    _, _, ohb = _argmax_rows(comb, lin)
    xb = _ext(x, ohb)
    yb = _ext(y, ohb)
    de = jnp.sqrt((xb - tx) ** 2 + (yb - ty) ** 2 + 1e-12)

    scores = comb
    scal = jnp.where(lane8 == 0, partial_loss[:, None], 0.0)
    scal = scal + jnp.where(lane8 == 1, de[:, None], 0.0)
    for k in range(EVAL_NUM):
        _, idx, oh = _argmax_rows(scores, lin)
        pk = _ext(comb, oh)
        xi = _ext(x, oh)
        yi = _ext(y, oh)
        idxs_acc = idxs_acc + jnp.where(sub16 == (POS_NUM + k), idx[None, :], 0)
        scal = scal + jnp.where(lane8 == (2 + k), pk[:, None], 0.0)
        d2 = (x - xi[:, None]) ** 2 + (y - yi[:, None]) ** 2
        scores = jnp.where(d2 + 1e-12 < 4.0, NEG, scores)

    scal_ref[:, :] = scal
    idx_ref[:, :] = idxs_acc


def _gather_body(idx_ref, tt_ref, rows_ref):
    b = pl.program_id(0)
    lane_b = lax.broadcasted_iota(jnp.int32, (16, B), 1)
    idxcol = jnp.sum(jnp.where(lane_b == b, idx_ref[:, :], 0), axis=1)
    lane_n = lax.broadcasted_iota(jnp.int32, (16, N), 1)
    oneh = (lane_n == idxcol[:, None]).astype(jnp.float32)
    for t in range(T):
        plane = tt_ref[0, t]
        s = lax.dot_general(oneh, plane, (((1,), (1,)), ((), ())),
                            preferred_element_type=jnp.float32)
        rows_ref[0, :, t, :] = s


def _finish_body(rows_ref, gt_ref, scal_ref, out_ref):
    rows = rows_ref[:, :, :, :]
    gt = gt_ref[:, :, :]
    sl1 = _smooth_l1_sum(rows - gt[:, None, :, :])
    rmask = lax.broadcasted_iota(jnp.int32, (B, 16, 1, 1), 1) < POS_NUM
    traj_loss = jnp.sum(jnp.where(rmask, sl1, 0.0), axis=(1, 2, 3)) / (POS_NUM * D)
    total = scal_ref[:, 0] + traj_loss
    out_ref[:, :] = jnp.reshape(jnp.sum(total) / B, (1, 1))


@jax.jit
def kernel(outputs_coord, outputs_class, outputs_traj, outputs_centerness,
           gt_points):
    interpret = False
    sel = jnp.array([[1.0, 0.0], [0.0, 1.0]], jnp.float32)
    x = jnp.sum(outputs_coord * sel[0], axis=-1)
    y = jnp.sum(outputs_coord * sel[1], axis=-1)
    tp = gt_points[:, -1, :]

    scal, gidx = pl.pallas_call(
        _compute_body,
        out_shape=(
            jax.ShapeDtypeStruct((B, 8), jnp.float32),
            jax.ShapeDtypeStruct((16, B), jnp.int32),
        ),
        interpret=interpret,
    )(x, y, outputs_class, outputs_centerness, tp)

    tt = jnp.transpose(outputs_traj, (0, 2, 3, 1))
    rows = pl.pallas_call(
        _gather_body,
        grid=(B,),
        in_specs=[
            pl.BlockSpec((16, B), lambda b: (0, 0)),
            pl.BlockSpec((1, T, 2, N), lambda b: (b, 0, 0, 0)),
        ],
        out_specs=pl.BlockSpec((1, 16, T, 2), lambda b: (b, 0, 0, 0)),
        out_shape=jax.ShapeDtypeStruct((B, 16, T, 2), jnp.float32),
        interpret=interpret,
    )(gidx, tt)

    loss = pl.pallas_call(
        _finish_body,
        out_shape=jax.ShapeDtypeStruct((1, 1), jnp.float32),
        interpret=interpret,
    )(rows, gt_points, scal)

    return (loss[0, 0], scal[:, 1], rows[:, POS_NUM:], scal[:, 2:8])


# speedup vs baseline: 41.1347x; 2.9847x over previous
import jax
import jax.numpy as jnp
from jax import lax
from jax.experimental import pallas as pl
from jax.experimental.pallas import tpu as pltpu

B, N, T = 16, 20000, 30
EVAL_NUM = 6
POS_NUM = 10
D = T * 2
BIG = 2**30
NEG = -jnp.inf
HALF = 8


def _argmax_rows(work, lin):
    m = jnp.max(work, axis=1)
    cand = jnp.where(work == m[:, None], lin, BIG)
    idx = jnp.min(cand, axis=1)
    oh = lin == idx[:, None]
    return m, idx, oh


def _ext(arr, oh):
    return jnp.sum(jnp.where(oh, arr, 0.0), axis=1)


def _smooth_l1(diff):
    d = jnp.abs(diff)
    return jnp.where(d < 1.0, 0.5 * d * d, d - 0.5)


def _compute_body(x_ref, y_ref, cls_ref, cent_ref, tp_ref,
                  scal_ref, idxt_ref, idx3_ref):
    x = x_ref[:, :]
    y = y_ref[:, :]
    cls = cls_ref[:, :]
    cent = cent_ref[:, :]
    tx = tp_ref[:, 0]
    ty = tp_ref[:, 1]

    lin = lax.broadcasted_iota(jnp.int32, (B, N), 1)
    sub16 = lax.broadcasted_iota(jnp.int32, (16, B), 0)
    lane16 = lax.broadcasted_iota(jnp.int32, (B, 16), 1)
    lane8 = lax.broadcasted_iota(jnp.int32, (B, 8), 1)

    work = cls
    idxs_t = jnp.zeros((16, B), jnp.int32)
    idxs_l = jnp.zeros((B, 16), jnp.int32)
    point_sum = jnp.zeros((B,), jnp.float32)
    class_sum = jnp.zeros((B,), jnp.float32)
    cent_sum = jnp.zeros((B,), jnp.float32)
    neg_top_sum = jnp.zeros((B,), jnp.float32)
    for k in range(POS_NUM):
        v, idx, oh = _argmax_rows(work, lin)
        xk = _ext(x, oh)
        yk = _ext(y, oh)
        ck = _ext(cent, oh)
        idxs_t = idxs_t + jnp.where(sub16 == k, idx[None, :], 0)
        idxs_l = idxs_l + jnp.where(lane16 == k, idx[:, None], 0)
        point_sum = point_sum + _smooth_l1(xk - tx) + _smooth_l1(yk - ty)
        class_sum = class_sum - jnp.log(jnp.clip(v, 1e-7, 1.0 - 1e-7))
        dk = jnp.sqrt((xk - tx) ** 2 + (yk - ty) ** 2 + 1e-12)
        tgt = jnp.where(dk >= 2.0, 0.0, 1.0 - jnp.sqrt(dk / 2.0))
        p = jnp.clip(ck, 1e-7, 1.0 - 1e-7)
        cent_sum = cent_sum - (tgt * jnp.log(p) + (1.0 - tgt) * jnp.log(1.0 - p))
        neg_top_sum = neg_top_sum - jnp.log(jnp.clip(1.0 - v, 1e-7, 1.0))
        work = jnp.where(oh, NEG, work)

    point_loss = point_sum / (POS_NUM * 2)
    class_loss = class_sum / POS_NUM
    centerness_loss = cent_sum / POS_NUM
    neg_all = jnp.sum(-jnp.log(jnp.clip(1.0 - cls, 1e-7, 1.0)), axis=1)
    neg_class_loss = (neg_all - neg_top_sum) / (N - POS_NUM)
    partial_loss = point_loss + class_loss + centerness_loss + neg_class_loss

    comb = cls * cent
    _, _, ohb = _argmax_rows(comb, lin)
    xb = _ext(x, ohb)
    yb = _ext(y, ohb)
    de = jnp.sqrt((xb - tx) ** 2 + (yb - ty) ** 2 + 1e-12)

    scores = comb
    scal = jnp.where(lane8 == 0, partial_loss[:, None], 0.0)
    scal = scal + jnp.where(lane8 == 1, de[:, None], 0.0)
    for k in range(EVAL_NUM):
        _, idx, oh = _argmax_rows(scores, lin)
        pk = _ext(comb, oh)
        xi = _ext(x, oh)
        yi = _ext(y, oh)
        idxs_t = idxs_t + jnp.where(sub16 == (POS_NUM + k), idx[None, :], 0)
        idxs_l = idxs_l + jnp.where(lane16 == (POS_NUM + k), idx[:, None], 0)
        scal = scal + jnp.where(lane8 == (2 + k), pk[:, None], 0.0)
        d2 = (x - xi[:, None]) ** 2 + (y - yi[:, None]) ** 2
        scores = jnp.where(d2 + 1e-12 < 4.0, NEG, scores)

    scal_ref[:, :] = scal
    idxt_ref[:, :] = idxs_t
    idx3_ref[:, 0, :] = idxs_l


def _gather_body(idx3_ref, idxt_ref, tt_ref, gt_ref, scal_ref,
                 rows_ref, loss_ref, chunks, sem):
    lane128 = lax.broadcasted_iota(jnp.int32, (16, T, 1, 128), 3)
    lane_b = lax.broadcasted_iota(jnp.int32, (16, B), 1)
    iota_b = lax.broadcasted_iota(jnp.int32, (B,), 0)
    jmask = (lax.broadcasted_iota(jnp.int32, (16, T, 2), 0) < POS_NUM)
    idxt = idxt_ref[:, :]
    tl_vec = jnp.zeros((B,), jnp.float32)
    for half in range(B // HALF):
        copies = []
        for s in range(HALF):
            b = half * HALF + s
            for j in range(16):
                idx = idx3_ref[b, 0, j]
                start = (idx // 128) * 128
                c = pltpu.make_async_copy(
                    tt_ref.at[b, :, :, pl.ds(start, 128)],
                    chunks.at[s * 16 + j], sem)
                c.start()
                copies.append(c)
        for c in copies:
            c.wait()
        for s in range(HALF):
            b = half * HALF + s
            mvec = jnp.sum(jnp.where(lane_b == b, idxt, 0), axis=1) % 128
            mask4 = lane128 == mvec[:, None, None, None]
            chunk = chunks[s * 16:(s + 1) * 16]
            g = jnp.sum(jnp.where(mask4, chunk, 0.0), axis=3)
            rows_ref[b] = g
            sl1 = _smooth_l1(g - gt_ref[b][None, :, :])
            tl_b = jnp.sum(jnp.where(jmask, sl1, 0.0)) / (POS_NUM * D)
            tl_vec = tl_vec + jnp.where(iota_b == b, tl_b, 0.0)
    total = scal_ref[:, 0] + tl_vec
    loss_ref[:, :] = jnp.reshape(jnp.sum(total) / B, (1, 1))


@jax.jit
def kernel(outputs_coord, outputs_class, outputs_traj, outputs_centerness,
           gt_points):
    interpret = False
    sel = jnp.array([[1.0, 0.0], [0.0, 1.0]], jnp.float32)
    x = jnp.sum(outputs_coord * sel[0], axis=-1)
    y = jnp.sum(outputs_coord * sel[1], axis=-1)
    tp = gt_points[:, -1, :]

    scal, gidxt, gidx3 = pl.pallas_call(
        _compute_body,
        out_shape=(
            jax.ShapeDtypeStruct((B, 8), jnp.float32),
            jax.ShapeDtypeStruct((16, B), jnp.int32),
            jax.ShapeDtypeStruct((B, 1, 16), jnp.int32),
        ),
        interpret=interpret,
    )(x, y, outputs_class, outputs_centerness, tp)

    tt = jnp.transpose(outputs_traj, (0, 2, 3, 1))
    rows, loss = pl.pallas_call(
        _gather_body,
        in_specs=[
            pl.BlockSpec(memory_space=pltpu.SMEM),
            pl.BlockSpec((16, B), lambda: (0, 0)),
            pl.BlockSpec(memory_space=pl.MemorySpace.ANY),
            pl.BlockSpec((B, T, 2), lambda: (0, 0, 0)),
            pl.BlockSpec((B, 8), lambda: (0, 0)),
        ],
        out_specs=(
            pl.BlockSpec((B, 16, T, 2), lambda: (0, 0, 0, 0)),
            pl.BlockSpec((1, 1), lambda: (0, 0)),
        ),
        out_shape=(
            jax.ShapeDtypeStruct((B, 16, T, 2), jnp.float32),
            jax.ShapeDtypeStruct((1, 1), jnp.float32),
        ),
        scratch_shapes=[
            pltpu.VMEM((HALF * 16, T, 2, 128), jnp.float32),
            pltpu.SemaphoreType.DMA,
        ],
        interpret=interpret,
    )(gidx3, gidxt, tt, gt_points, scal)

    return (loss[0, 0], scal[:, 1], rows[:, POS_NUM:], scal[:, 2:8])


# speedup vs baseline: 42.8943x; 1.0428x over previous
import jax
import jax.numpy as jnp
from jax import lax
from jax.experimental import pallas as pl
from jax.experimental.pallas import tpu as pltpu

B, N, T = 16, 20000, 30
EVAL_NUM = 6
POS_NUM = 10
D = T * 2
BIG = 2**30
NEG = -jnp.inf
HALF = 8


def _argmax_rows(work, lin):
    m = jnp.max(work, axis=1)
    cand = jnp.where(work == m[:, None], lin, BIG)
    idx = jnp.min(cand, axis=1)
    oh = lin == idx[:, None]
    return m, idx, oh


def _ext(arr, oh):
    return jnp.sum(jnp.where(oh, arr, 0.0), axis=1)


def _smooth_l1(diff):
    d = jnp.abs(diff)
    return jnp.where(d < 1.0, 0.5 * d * d, d - 0.5)


def _compute_body(tc_ref, cls_ref, cent_ref, tp_ref,
                  scal_ref, idxt_ref, idx3_ref):
    x = tc_ref[:, 0, :]
    y = tc_ref[:, 1, :]
    cls = cls_ref[:, :]
    cent = cent_ref[:, :]
    tx = tp_ref[:, 0]
    ty = tp_ref[:, 1]

    lin = lax.broadcasted_iota(jnp.int32, (B, N), 1)
    sub16 = lax.broadcasted_iota(jnp.int32, (16, B), 0)
    lane16 = lax.broadcasted_iota(jnp.int32, (B, 16), 1)
    lane8 = lax.broadcasted_iota(jnp.int32, (B, 8), 1)

    work = cls
    idxs_t = jnp.zeros((16, B), jnp.int32)
    idxs_l = jnp.zeros((B, 16), jnp.int32)
    point_sum = jnp.zeros((B,), jnp.float32)
    class_sum = jnp.zeros((B,), jnp.float32)
    cent_sum = jnp.zeros((B,), jnp.float32)
    neg_top_sum = jnp.zeros((B,), jnp.float32)
    for k in range(POS_NUM):
        v, idx, oh = _argmax_rows(work, lin)
        xk = _ext(x, oh)
        yk = _ext(y, oh)
        ck = _ext(cent, oh)
        idxs_t = idxs_t + jnp.where(sub16 == k, idx[None, :], 0)
        idxs_l = idxs_l + jnp.where(lane16 == k, idx[:, None], 0)
        point_sum = point_sum + _smooth_l1(xk - tx) + _smooth_l1(yk - ty)
        class_sum = class_sum - jnp.log(jnp.clip(v, 1e-7, 1.0 - 1e-7))
        dk = jnp.sqrt((xk - tx) ** 2 + (yk - ty) ** 2 + 1e-12)
        tgt = jnp.where(dk >= 2.0, 0.0, 1.0 - jnp.sqrt(dk / 2.0))
        p = jnp.clip(ck, 1e-7, 1.0 - 1e-7)
        cent_sum = cent_sum - (tgt * jnp.log(p) + (1.0 - tgt) * jnp.log(1.0 - p))
        neg_top_sum = neg_top_sum - jnp.log(jnp.clip(1.0 - v, 1e-7, 1.0))
        work = jnp.where(oh, NEG, work)

    point_loss = point_sum / (POS_NUM * 2)
    class_loss = class_sum / POS_NUM
    centerness_loss = cent_sum / POS_NUM
    neg_all = jnp.sum(-jnp.log(jnp.clip(1.0 - cls, 1e-7, 1.0)), axis=1)
    neg_class_loss = (neg_all - neg_top_sum) / (N - POS_NUM)
    partial_loss = point_loss + class_loss + centerness_loss + neg_class_loss

    comb = cls * cent
    _, _, ohb = _argmax_rows(comb, lin)
    xb = _ext(x, ohb)
    yb = _ext(y, ohb)
    de = jnp.sqrt((xb - tx) ** 2 + (yb - ty) ** 2 + 1e-12)

    scores = comb
    scal = jnp.where(lane8 == 0, partial_loss[:, None], 0.0)
    scal = scal + jnp.where(lane8 == 1, de[:, None], 0.0)
    for k in range(EVAL_NUM):
        _, idx, oh = _argmax_rows(scores, lin)
        pk = _ext(comb, oh)
        xi = _ext(x, oh)
        yi = _ext(y, oh)
        idxs_t = idxs_t + jnp.where(sub16 == (POS_NUM + k), idx[None, :], 0)
        idxs_l = idxs_l + jnp.where(lane16 == (POS_NUM + k), idx[:, None], 0)
        scal = scal + jnp.where(lane8 == (2 + k), pk[:, None], 0.0)
        d2 = (x - xi[:, None]) ** 2 + (y - yi[:, None]) ** 2
        scores = jnp.where(d2 + 1e-12 < 4.0, NEG, scores)

    scal_ref[:, :] = scal
    idxt_ref[:, :] = idxs_t
    idx3_ref[:, 0, :] = idxs_l


def _gather_body(idx3_ref, idxt_ref, tt_ref, gt_ref, scal_ref,
                 rows_ref, loss_ref, chunks, sem):
    lane128 = lax.broadcasted_iota(jnp.int32, (16, T, 1, 128), 3)
    lane_b = lax.broadcasted_iota(jnp.int32, (16, B), 1)
    iota_b = lax.broadcasted_iota(jnp.int32, (B,), 0)
    jmask = (lax.broadcasted_iota(jnp.int32, (16, T, 2), 0) < POS_NUM)
    idxt = idxt_ref[:, :]
    tl_vec = jnp.zeros((B,), jnp.float32)
    for half in range(B // HALF):
        copies = []
        for s in range(HALF):
            b = half * HALF + s
            for j in range(16):
                idx = idx3_ref[b, 0, j]
                start = (idx // 128) * 128
                c = pltpu.make_async_copy(
                    tt_ref.at[b, :, :, pl.ds(start, 128)],
                    chunks.at[s * 16 + j], sem)
                c.start()
                copies.append(c)
        for c in copies:
            c.wait()
        for s in range(HALF):
            b = half * HALF + s
            mvec = jnp.sum(jnp.where(lane_b == b, idxt, 0), axis=1) % 128
            mask4 = lane128 == mvec[:, None, None, None]
            chunk = chunks[s * 16:(s + 1) * 16]
            g = jnp.sum(jnp.where(mask4, chunk, 0.0), axis=3)
            rows_ref[b] = g
            sl1 = _smooth_l1(g - gt_ref[b][None, :, :])
            tl_b = jnp.sum(jnp.where(jmask, sl1, 0.0)) / (POS_NUM * D)
            tl_vec = tl_vec + jnp.where(iota_b == b, tl_b, 0.0)
    total = scal_ref[:, 0] + tl_vec
    loss_ref[:, :] = jnp.reshape(jnp.sum(total) / B, (1, 1))


@jax.jit
def kernel(outputs_coord, outputs_class, outputs_traj, outputs_centerness,
           gt_points):
    interpret = False
    tc = jnp.transpose(outputs_coord, (0, 2, 1))
    tp = gt_points[:, -1, :]

    scal, gidxt, gidx3 = pl.pallas_call(
        _compute_body,
        out_shape=(
            jax.ShapeDtypeStruct((B, 8), jnp.float32),
            jax.ShapeDtypeStruct((16, B), jnp.int32),
            jax.ShapeDtypeStruct((B, 1, 16), jnp.int32),
        ),
        interpret=interpret,
    )(tc, outputs_class, outputs_centerness, tp)

    tt = jnp.transpose(outputs_traj, (0, 2, 3, 1))
    rows, loss = pl.pallas_call(
        _gather_body,
        in_specs=[
            pl.BlockSpec(memory_space=pltpu.SMEM),
            pl.BlockSpec((16, B), lambda: (0, 0)),
            pl.BlockSpec(memory_space=pl.MemorySpace.ANY),
            pl.BlockSpec((B, T, 2), lambda: (0, 0, 0)),
            pl.BlockSpec((B, 8), lambda: (0, 0)),
        ],
        out_specs=(
            pl.BlockSpec((B, 16, T, 2), lambda: (0, 0, 0, 0)),
            pl.BlockSpec((1, 1), lambda: (0, 0)),
        ),
        out_shape=(
            jax.ShapeDtypeStruct((B, 16, T, 2), jnp.float32),
            jax.ShapeDtypeStruct((1, 1), jnp.float32),
        ),
        scratch_shapes=[
            pltpu.VMEM((HALF * 16, T, 2, 128), jnp.float32),
            pltpu.SemaphoreType.DMA,
        ],
        interpret=interpret,
    )(gidx3, gidxt, tt, gt_points, scal)

    return (loss[0, 0], scal[:, 1], rows[:, POS_NUM:], scal[:, 2:8])


# speedup vs baseline: 44.3860x; 1.0348x over previous
import jax
import jax.numpy as jnp
from jax import lax
from jax.experimental import pallas as pl
from jax.experimental.pallas import tpu as pltpu

B, N, T = 16, 20000, 30
EVAL_NUM = 6
POS_NUM = 10
D = T * 2
BIG = 2**30
NEG = -jnp.inf
QUARTER = 4


def _argmax_rows(work, lin):
    m = jnp.max(work, axis=1)
    cand = jnp.where(work == m[:, None], lin, BIG)
    idx = jnp.min(cand, axis=1)
    oh = lin == idx[:, None]
    return m, idx, oh


def _ext(arr, oh):
    return jnp.sum(jnp.where(oh, arr, 0.0), axis=1)


def _smooth_l1(diff):
    d = jnp.abs(diff)
    return jnp.where(d < 1.0, 0.5 * d * d, d - 0.5)


def _body(tc_ref, cls_ref, cent_ref, tp_ref, tt_ref, gt_ref,
          scal_ref, rows_ref, loss_ref,
          idxv_ref, idxs_ref, chunks, sem, sem2, semi):
    x = tc_ref[:, 0, :]
    y = tc_ref[:, 1, :]
    cls = cls_ref[:, :]
    cent = cent_ref[:, :]
    tx = tp_ref[:, 0]
    ty = tp_ref[:, 1]

    lin = lax.broadcasted_iota(jnp.int32, (B, N), 1)
    sub16 = lax.broadcasted_iota(jnp.int32, (16, B), 0)
    lane16 = lax.broadcasted_iota(jnp.int32, (B, 16), 1)
    lane8 = lax.broadcasted_iota(jnp.int32, (B, 8), 1)

    work = cls
    idxs_t = jnp.zeros((16, B), jnp.int32)
    idxs_l = jnp.zeros((B, 16), jnp.int32)
    point_sum = jnp.zeros((B,), jnp.float32)
    class_sum = jnp.zeros((B,), jnp.float32)
    cent_sum = jnp.zeros((B,), jnp.float32)
    neg_top_sum = jnp.zeros((B,), jnp.float32)
    for k in range(POS_NUM):
        v, idx, oh = _argmax_rows(work, lin)
        xk = _ext(x, oh)
        yk = _ext(y, oh)
        ck = _ext(cent, oh)
        idxs_t = idxs_t + jnp.where(sub16 == k, idx[None, :], 0)
        idxs_l = idxs_l + jnp.where(lane16 == k, idx[:, None], 0)
        point_sum = point_sum + _smooth_l1(xk - tx) + _smooth_l1(yk - ty)
        class_sum = class_sum - jnp.log(jnp.clip(v, 1e-7, 1.0 - 1e-7))
        dk = jnp.sqrt((xk - tx) ** 2 + (yk - ty) ** 2 + 1e-12)
        tgt = jnp.where(dk >= 2.0, 0.0, 1.0 - jnp.sqrt(dk / 2.0))
        p = jnp.clip(ck, 1e-7, 1.0 - 1e-7)
        cent_sum = cent_sum - (tgt * jnp.log(p) + (1.0 - tgt) * jnp.log(1.0 - p))
        neg_top_sum = neg_top_sum - jnp.log(jnp.clip(1.0 - v, 1e-7, 1.0))
        work = jnp.where(oh, NEG, work)

    point_loss = point_sum / (POS_NUM * 2)
    class_loss = class_sum / POS_NUM
    centerness_loss = cent_sum / POS_NUM
    neg_all = jnp.sum(-jnp.log(jnp.clip(1.0 - cls, 1e-7, 1.0)), axis=1)
    neg_class_loss = (neg_all - neg_top_sum) / (N - POS_NUM)
    partial_loss = point_loss + class_loss + centerness_loss + neg_class_loss

    comb = cls * cent
    scores = comb
    scal = jnp.where(lane8 == 0, partial_loss[:, None], 0.0)
    for k in range(EVAL_NUM):
        _, idx, oh = _argmax_rows(scores, lin)
        pk = _ext(comb, oh)
        xi = _ext(x, oh)
        yi = _ext(y, oh)
        idxs_t = idxs_t + jnp.where(sub16 == (POS_NUM + k), idx[None, :], 0)
        idxs_l = idxs_l + jnp.where(lane16 == (POS_NUM + k), idx[:, None], 0)
        scal = scal + jnp.where(lane8 == (2 + k), pk[:, None], 0.0)
        if k == 0:
            de = jnp.sqrt((xi - tx) ** 2 + (yi - ty) ** 2 + 1e-12)
            scal = scal + jnp.where(lane8 == 1, de[:, None], 0.0)
        d2 = (x - xi[:, None]) ** 2 + (y - yi[:, None]) ** 2
        scores = jnp.where(d2 + 1e-12 < 4.0, NEG, scores)

    idxv_ref[:, :] = idxs_l
    cp = pltpu.make_async_copy(idxv_ref, idxs_ref, semi)
    cp.start()
    cp.wait()

    lane128 = lax.broadcasted_iota(jnp.int32, (16, T, 1, 128), 3)
    lane_b = lax.broadcasted_iota(jnp.int32, (16, B), 1)
    iota_b = lax.broadcasted_iota(jnp.int32, (B,), 0)
    jmask = (lax.broadcasted_iota(jnp.int32, (16, T, 2), 0) < POS_NUM)
    tl_vec = jnp.zeros((B,), jnp.float32)
    n_rounds = B // QUARTER
    sems = [sem, sem2]

    def issue(r):
        buf = r % 2
        for s in range(QUARTER):
            b = r * QUARTER + s
            for j in range(16):
                idx = idxs_ref[b, j]
                start = (idx // 128) * 128
                pltpu.make_async_copy(
                    tt_ref.at[b, :, :, pl.ds(start, 128)],
                    chunks.at[buf, s * 16 + j], sems[buf]).start()

    def drain(r):
        buf = r % 2
        for s in range(QUARTER):
            for j in range(16):
                pltpu.make_async_copy(
                    tt_ref.at[0, :, :, pl.ds(0, 128)],
                    chunks.at[buf, s * 16 + j], sems[buf]).wait()

    issue(0)
    issue(1)
    for r in range(n_rounds):
        drain(r)
        buf = r % 2
        for s in range(QUARTER):
            b = r * QUARTER + s
            mvec = jnp.sum(jnp.where(lane_b == b, idxs_t, 0), axis=1) % 128
            mask4 = lane128 == mvec[:, None, None, None]
            chunk = chunks[buf, s * 16:(s + 1) * 16]
            g = jnp.sum(jnp.where(mask4, chunk, 0.0), axis=3)
            rows_ref[b] = g
            sl1 = _smooth_l1(g - gt_ref[b][None, :, :])
            tl_b = jnp.sum(jnp.where(jmask, sl1, 0.0)) / (POS_NUM * D)
            tl_vec = tl_vec + jnp.where(iota_b == b, tl_b, 0.0)
        if r + 2 < n_rounds:
            issue(r + 2)
    total = scal[:, 0] + tl_vec
    loss_ref[:, :] = jnp.reshape(jnp.sum(total) / B, (1, 1))
    scal_ref[:, :] = scal


@jax.jit
def kernel(outputs_coord, outputs_class, outputs_traj, outputs_centerness,
           gt_points):
    interpret = False
    tc = jnp.transpose(outputs_coord, (0, 2, 1))
    tt = jnp.transpose(outputs_traj, (0, 2, 3, 1))
    tp = gt_points[:, -1, :]

    scal, rows, loss = pl.pallas_call(
        _body,
        in_specs=[
            pl.BlockSpec((B, 2, N), lambda: (0, 0, 0)),
            pl.BlockSpec((B, N), lambda: (0, 0)),
            pl.BlockSpec((B, N), lambda: (0, 0)),
            pl.BlockSpec((B, 2), lambda: (0, 0)),
            pl.BlockSpec(memory_space=pl.MemorySpace.ANY),
            pl.BlockSpec((B, T, 2), lambda: (0, 0, 0)),
        ],
        out_specs=(
            pl.BlockSpec((B, 8), lambda: (0, 0)),
            pl.BlockSpec((B, 16, T, 2), lambda: (0, 0, 0, 0)),
            pl.BlockSpec((1, 1), lambda: (0, 0)),
        ),
        out_shape=(
            jax.ShapeDtypeStruct((B, 8), jnp.float32),
            jax.ShapeDtypeStruct((B, 16, T, 2), jnp.float32),
            jax.ShapeDtypeStruct((1, 1), jnp.float32),
        ),
        scratch_shapes=[
            pltpu.VMEM((B, 16), jnp.int32),
            pltpu.SMEM((B, 16), jnp.int32),
            pltpu.VMEM((2, QUARTER * 16, T, 2, 128), jnp.float32),
            pltpu.SemaphoreType.DMA,
            pltpu.SemaphoreType.DMA,
            pltpu.SemaphoreType.DMA,
        ],
        interpret=interpret,
    )(tc, outputs_class, outputs_centerness, tp, tt, gt_points)

    return (loss[0, 0], scal[:, 1], rows[:, POS_NUM:], scal[:, 2:8])


# speedup vs baseline: 58.1845x; 1.3109x over previous
import jax
import jax.numpy as jnp
from jax import lax
from jax.experimental import pallas as pl
from jax.experimental.pallas import tpu as pltpu

B, N, T = 16, 20000, 30
EVAL_NUM = 6
POS_NUM = 10
D = T * 2
BIG = 2**30
NEG = -jnp.inf
QUARTER = 4


def _argmax_rows(work, lin):
    m = jnp.max(work, axis=1)
    cand = jnp.where(work == m[:, None], lin, BIG)
    idx = jnp.min(cand, axis=1)
    oh = lin == idx[:, None]
    return m, idx, oh


def _ext(arr, oh):
    return jnp.sum(jnp.where(oh, arr, 0.0), axis=1)


def _smooth_l1(diff):
    d = jnp.abs(diff)
    return jnp.where(d < 1.0, 0.5 * d * d, d - 0.5)


def _body(tc_ref, cls_ref, cent_ref, tt_ref, gtt_ref,
          scal_ref, rows_ref, loss_ref,
          idxv_ref, idxs_ref, chunks, sem, sem2, semi):
    x = tc_ref[:, 0, :]
    y = tc_ref[:, 1, :]
    cls = cls_ref[:, :]
    cent = cent_ref[:, :]
    tx = gtt_ref[:, 0, T - 1]
    ty = gtt_ref[:, 1, T - 1]

    lin = lax.broadcasted_iota(jnp.int32, (B, N), 1)
    sub16 = lax.broadcasted_iota(jnp.int32, (16, B), 0)
    lane16 = lax.broadcasted_iota(jnp.int32, (B, 16), 1)
    lane8 = lax.broadcasted_iota(jnp.int32, (B, 8), 1)

    work = cls
    idxs_t = jnp.zeros((16, B), jnp.int32)
    idxs_l = jnp.zeros((B, 16), jnp.int32)
    point_sum = jnp.zeros((B,), jnp.float32)
    class_sum = jnp.zeros((B,), jnp.float32)
    cent_sum = jnp.zeros((B,), jnp.float32)
    neg_top_sum = jnp.zeros((B,), jnp.float32)
    for k in range(POS_NUM):
        v, idx, oh = _argmax_rows(work, lin)
        xk = _ext(x, oh)
        yk = _ext(y, oh)
        ck = _ext(cent, oh)
        idxs_t = idxs_t + jnp.where(sub16 == k, idx[None, :], 0)
        idxs_l = idxs_l + jnp.where(lane16 == k, idx[:, None], 0)
        point_sum = point_sum + _smooth_l1(xk - tx) + _smooth_l1(yk - ty)
        class_sum = class_sum - jnp.log(jnp.clip(v, 1e-7, 1.0 - 1e-7))
        dk = jnp.sqrt((xk - tx) ** 2 + (yk - ty) ** 2 + 1e-12)
        tgt = jnp.where(dk >= 2.0, 0.0, 1.0 - jnp.sqrt(dk / 2.0))
        p = jnp.clip(ck, 1e-7, 1.0 - 1e-7)
        cent_sum = cent_sum - (tgt * jnp.log(p) + (1.0 - tgt) * jnp.log(1.0 - p))
        neg_top_sum = neg_top_sum - jnp.log(jnp.clip(1.0 - v, 1e-7, 1.0))
        work = jnp.where(oh, NEG, work)

    point_loss = point_sum / (POS_NUM * 2)
    class_loss = class_sum / POS_NUM
    centerness_loss = cent_sum / POS_NUM
    neg_all = jnp.sum(-jnp.log(jnp.clip(1.0 - cls, 1e-7, 1.0)), axis=1)
    neg_class_loss = (neg_all - neg_top_sum) / (N - POS_NUM)
    partial_loss = point_loss + class_loss + centerness_loss + neg_class_loss

    comb = cls * cent
    scores = comb
    scal = jnp.where(lane8 == 0, partial_loss[:, None], 0.0)
    for k in range(EVAL_NUM):
        _, idx, oh = _argmax_rows(scores, lin)
        pk = _ext(comb, oh)
        xi = _ext(x, oh)
        yi = _ext(y, oh)
        idxs_t = idxs_t + jnp.where(sub16 == (POS_NUM + k), idx[None, :], 0)
        idxs_l = idxs_l + jnp.where(lane16 == (POS_NUM + k), idx[:, None], 0)
        scal = scal + jnp.where(lane8 == (2 + k), pk[:, None], 0.0)
        if k == 0:
            de = jnp.sqrt((xi - tx) ** 2 + (yi - ty) ** 2 + 1e-12)
            scal = scal + jnp.where(lane8 == 1, de[:, None], 0.0)
        d2 = (x - xi[:, None]) ** 2 + (y - yi[:, None]) ** 2
        scores = jnp.where(d2 + 1e-12 < 4.0, NEG, scores)

    idxv_ref[:, :] = idxs_l
    cp = pltpu.make_async_copy(idxv_ref, idxs_ref, semi)
    cp.start()
    cp.wait()

    lane128 = lax.broadcasted_iota(jnp.int32, (16, T, 1, 128), 3)
    lane_b = lax.broadcasted_iota(jnp.int32, (16, B), 1)
    iota_b = lax.broadcasted_iota(jnp.int32, (B,), 0)
    jmask = (lax.broadcasted_iota(jnp.int32, (16, 2, T), 0) < POS_NUM)
    tl_vec = jnp.zeros((B,), jnp.float32)
    n_rounds = B // QUARTER
    sems = [sem, sem2]

    def issue(r):
        buf = r % 2
        for s in range(QUARTER):
            b = r * QUARTER + s
            for j in range(16):
                idx = idxs_ref[b, j]
                start = (idx // 128) * 128
                pltpu.make_async_copy(
                    tt_ref.at[b, :, :, pl.ds(start, 128)],
                    chunks.at[buf, s * 16 + j], sems[buf]).start()

    def drain(r):
        buf = r % 2
        for s in range(QUARTER):
            for j in range(16):
                pltpu.make_async_copy(
                    tt_ref.at[0, :, :, pl.ds(0, 128)],
                    chunks.at[buf, s * 16 + j], sems[buf]).wait()

    issue(0)
    issue(1)
    for r in range(n_rounds):
        drain(r)
        buf = r % 2
        for s in range(QUARTER):
            b = r * QUARTER + s
            mvec = jnp.sum(jnp.where(lane_b == b, idxs_t, 0), axis=1) % 128
            mask4 = lane128 == mvec[:, None, None, None]
            chunk = chunks[buf, s * 16:(s + 1) * 16]
            g = jnp.sum(jnp.where(mask4, chunk, 0.0), axis=3)
            g_ct = jnp.transpose(g, (0, 2, 1))
            rows_ref[b] = g_ct
            sl1 = _smooth_l1(g_ct - gtt_ref[b][None, :, :])
            tl_b = jnp.sum(jnp.where(jmask, sl1, 0.0)) / (POS_NUM * D)
            tl_vec = tl_vec + jnp.where(iota_b == b, tl_b, 0.0)
        if r + 2 < n_rounds:
            issue(r + 2)
    total = scal[:, 0] + tl_vec
    loss_ref[:, :] = jnp.reshape(jnp.sum(total) / B, (1, 1))
    scal_ref[:, :] = scal


@jax.jit
def kernel(outputs_coord, outputs_class, outputs_traj, outputs_centerness,
           gt_points):
    interpret = False
    tc = jnp.transpose(outputs_coord, (0, 2, 1))
    tt = jnp.transpose(outputs_traj, (0, 2, 3, 1))
    gtt = jnp.transpose(gt_points, (0, 2, 1))

    scal, rows, loss = pl.pallas_call(
        _body,
        in_specs=[
            pl.BlockSpec((B, 2, N), lambda: (0, 0, 0)),
            pl.BlockSpec((B, N), lambda: (0, 0)),
            pl.BlockSpec((B, N), lambda: (0, 0)),
            pl.BlockSpec(memory_space=pl.MemorySpace.ANY),
            pl.BlockSpec((B, 2, T), lambda: (0, 0, 0)),
        ],
        out_specs=(
            pl.BlockSpec((B, 8), lambda: (0, 0)),
            pl.BlockSpec((B, 16, 2, T), lambda: (0, 0, 0, 0)),
            pl.BlockSpec((1, 1), lambda: (0, 0)),
        ),
        out_shape=(
            jax.ShapeDtypeStruct((B, 8), jnp.float32),
            jax.ShapeDtypeStruct((B, 16, 2, T), jnp.float32),
            jax.ShapeDtypeStruct((1, 1), jnp.float32),
        ),
        scratch_shapes=[
            pltpu.VMEM((B, 16), jnp.int32),
            pltpu.SMEM((B, 16), jnp.int32),
            pltpu.VMEM((2, QUARTER * 16, T, 2, 128), jnp.float32),
            pltpu.SemaphoreType.DMA,
            pltpu.SemaphoreType.DMA,
            pltpu.SemaphoreType.DMA,
        ],
        interpret=interpret,
    )(tc, outputs_class, outputs_centerness, tt, gtt)

    trajs = jnp.transpose(rows[:, POS_NUM:], (0, 1, 3, 2))
    return (loss[0, 0], scal[:, 1], trajs, scal[:, 2:8])
